# baseline (speedup 1.0000x reference)
import jax
import jax.numpy as jnp
from jax import lax
from jax.experimental import pallas as pl
from jax.experimental.pallas import tpu as pltpu

_N, _P, _G, _C = 8, 8732, 64, 21
_P_TILE = 640
_INT_MIN = -(2 ** 31)


def _stage1_body(pred_ref, gtx_ref, dfx_ref, lconf_ref, lloc_ref, many_ref,
                 mf_ref):
    predT = pred_ref[0].T
    logits = predT[4:25, :]
    m = jnp.max(logits, axis=0, keepdims=True)
    lse = jnp.log(jnp.sum(jnp.exp(logits - m), axis=0, keepdims=True)) + m
    logp = logits - lse

    gtx = gtx_ref[0]
    gclsT = gtx[:, 9:30].T

    d_l = dfx_ref[0:1, :]
    d_r = dfx_ref[1:2, :]
    d_b = dfx_ref[2:3, :]
    d_t = dfx_ref[3:4, :]
    d_area = dfx_ref[4:5, :]
    inv_dw = dfx_ref[5:6, :]
    inv_dh = dfx_ref[6:7, :]

    a_cx = predT[0:1, :] + dfx_ref[7:8, :]
    a_cy = predT[1:2, :] + dfx_ref[8:9, :]
    a_w = predT[2:3, :] + dfx_ref[9:10, :]
    a_h = predT[3:4, :] + dfx_ref[10:11, :]

    def sl1(x):
        ax = jnp.abs(x)
        mm = jnp.minimum(ax, 1.0)
        return (ax - mm) + (0.5 * mm) * mm

    acc = jnp.zeros((8, _P_TILE), jnp.float32)
    for c in range(_G // 8):
        sl = slice(c * 8, c * 8 + 8)
        g_cx = gtx[sl, 0:1]
        g_cy = gtx[sl, 1:2]
        g_l = gtx[sl, 2:3]
        g_r = gtx[sl, 3:4]
        g_b = gtx[sl, 4:5]
        g_t = gtx[sl, 5:6]
        g_area = gtx[sl, 6:7]
        log_gw = gtx[sl, 7:8]
        log_gh = gtx[sl, 8:9]
        w = jnp.maximum(jnp.minimum(g_r, d_r) - jnp.maximum(g_l, d_l), 0.0)
        h = jnp.maximum(jnp.minimum(g_t, d_t) - jnp.maximum(g_b, d_b), 0.0)
        inter = w * h
        match = (3.0 * inter) > (g_area + d_area)
        mf_ref[sl, :] = match.astype(jnp.float32)
        x1 = a_cx - g_cx * inv_dw
        x2 = a_cy - g_cy * inv_dh
        x3 = a_w - log_gw
        x4 = a_h - log_gh
        s = sl1(x1) + sl1(x2) + sl1(x3) + sl1(x4)
        acc = acc + jnp.where(match, s, 0.0)

    lloc = jnp.sum(acc, axis=0, keepdims=True)
    mm_ = jnp.dot(gclsT, mf_ref[...], preferred_element_type=jnp.float32)
    cnt = jnp.sum(mm_, axis=0, keepdims=True)
    lcp = jnp.sum(logp * mm_, axis=0, keepdims=True)
    lc = jnp.where(cnt > 0.0, -lcp, logp[0:1, :])

    lconf_ref[0] = lc
    lloc_ref[0] = lloc
    many_ref[0] = (cnt > 0.0).astype(jnp.float32)


def _stage2_body(lconf_ref, lloc_ref, many_ref, a_ref, out_ref):
    lc = lconf_ref[...]
    f = lax.bitcast_convert_type(lc, jnp.int32)
    imin = jnp.int32(_INT_MIN)
    ukey = jnp.where(f < 0, ~f, f ^ imin)
    skey = ukey ^ imin

    posf = jnp.sum(many_ref[...], axis=1, keepdims=True)
    pos_orig = posf.astype(jnp.int32)
    pos = jnp.maximum(pos_orig, 1)
    neg = jnp.maximum(jnp.minimum(_P - pos_orig, 3 * pos), 1)
    k_lo0 = neg
    k_hi0 = _P - pos + 1

    def body(i, carry):
        prefix_lo, k_lo, prefix_hi, k_hi, hmask, bitv = carry
        masked = ukey & hmask
        bit0 = (ukey & bitv) == 0
        cand_lo = (masked == prefix_lo) & bit0
        c0_lo = jnp.sum(cand_lo.astype(jnp.int32), axis=1, keepdims=True)
        take0_lo = k_lo <= c0_lo
        prefix_lo = jnp.where(take0_lo, prefix_lo, prefix_lo | bitv)
        k_lo = jnp.where(take0_lo, k_lo, k_lo - c0_lo)
        cand_hi = (masked == prefix_hi) & bit0
        c0_hi = jnp.sum(cand_hi.astype(jnp.int32), axis=1, keepdims=True)
        take0_hi = k_hi <= c0_hi
        prefix_hi = jnp.where(take0_hi, prefix_hi, prefix_hi | bitv)
        k_hi = jnp.where(take0_hi, k_hi, k_hi - c0_hi)
        hmask = hmask | bitv
        bitv = lax.shift_right_logical(bitv, 1)
        return prefix_lo, k_lo, prefix_hi, k_hi, hmask, bitv

    zeros = jnp.zeros((_N, 1), jnp.int32)
    prefix_lo, _, prefix_hi, _, _, _ = lax.fori_loop(
        0, 32, body, (zeros, k_lo0, zeros, k_hi0, jnp.int32(0), imin),
        unroll=4)

    skth_lo = prefix_lo ^ imin
    skth_hi = prefix_hi ^ imin
    valid = (skey < skth_lo) | (skey > skth_hi)
    contrib = jnp.where(valid, lloc_ref[...] + a_ref[0, 0] * jnp.abs(lc), 0.0)
    rows = jnp.sum(contrib, axis=1, keepdims=True) / pos.astype(jnp.float32)
    out_ref[...] = jnp.sum(rows, axis=0, keepdims=True) / float(_N)


def kernel(pred_bboxes, default_bboxes, gt_bboxes, a=1):
    dcx = default_bboxes[:, 0]
    dcy = default_bboxes[:, 1]
    dw = default_bboxes[:, 2]
    dh = default_bboxes[:, 3]
    dfx = jnp.stack([
        dcx - dw * 0.5, dcx + dw * 0.5, dcy - dh * 0.5, dcy + dh * 0.5,
        dw * dh, 1.0 / dw, 1.0 / dh, dcx / dw, dcy / dh,
        jnp.log(dw), jnp.log(dh)], axis=0)

    gcx = gt_bboxes[..., 0]
    gcy = gt_bboxes[..., 1]
    gw = gt_bboxes[..., 2]
    gh = gt_bboxes[..., 3]
    gtx = jnp.concatenate([
        jnp.stack([gcx, gcy, gcx - gw * 0.5, gcx + gw * 0.5,
                   gcy - gh * 0.5, gcy + gh * 0.5, gw * gh,
                   jnp.log(gw), jnp.log(gh)], axis=-1),
        gt_bboxes[..., 4:25],
        jnp.zeros((_N, _G, 2), jnp.float32)], axis=-1)

    n_tiles = (_P + _P_TILE - 1) // _P_TILE
    out2 = jax.ShapeDtypeStruct((_N, 1, _P), jnp.float32)
    lconf, lloc, many = pl.pallas_call(
        _stage1_body,
        grid=(_N, n_tiles),
        in_specs=[
            pl.BlockSpec((1, _P_TILE, 25), lambda n, t: (n, t, 0)),
            pl.BlockSpec((1, _G, 32), lambda n, t: (n, 0, 0)),
            pl.BlockSpec((11, _P_TILE), lambda n, t: (0, t)),
        ],
        out_specs=[
            pl.BlockSpec((1, 1, _P_TILE), lambda n, t: (n, 0, t)),
            pl.BlockSpec((1, 1, _P_TILE), lambda n, t: (n, 0, t)),
            pl.BlockSpec((1, 1, _P_TILE), lambda n, t: (n, 0, t)),
        ],
        out_shape=[out2, out2, out2],
        scratch_shapes=[pltpu.VMEM((_G, _P_TILE), jnp.float32)],
    )(pred_bboxes, gtx, dfx)

    a_arr = jnp.full((1, 1), a, jnp.float32)
    out = pl.pallas_call(
        _stage2_body,
        out_shape=jax.ShapeDtypeStruct((1, 1), jnp.float32),
    )(lconf.reshape(_N, _P), lloc.reshape(_N, _P),
      many.reshape(_N, _P), a_arr)
    return out[0, 0]


# speedup vs baseline: 1.4111x; 1.4111x over previous
import jax
import jax.numpy as jnp
from jax import lax
from jax.experimental import pallas as pl
from jax.experimental.pallas import tpu as pltpu

_N, _P, _G, _C = 8, 8732, 64, 21
_P_TILE = 1792
_INT_MIN = -(2 ** 31)


def _stage1_body(pred_ref, gtx_ref, dfx_ref, lconf_ref, lloc_ref, many_ref,
                 mf_ref):
    predT = pred_ref[0].T
    logits = predT[4:25, :]
    m = jnp.max(logits, axis=0, keepdims=True)
    lse = jnp.log(jnp.sum(jnp.exp(logits - m), axis=0, keepdims=True)) + m
    logp = logits - lse

    gtx = gtx_ref[0]
    gclsT = gtx[:, 9:30].T

    d_l = dfx_ref[0:1, :]
    d_r = dfx_ref[1:2, :]
    d_b = dfx_ref[2:3, :]
    d_t = dfx_ref[3:4, :]
    d_area = dfx_ref[4:5, :]
    inv_dw = dfx_ref[5:6, :]
    inv_dh = dfx_ref[6:7, :]

    a_cx = predT[0:1, :] + dfx_ref[7:8, :]
    a_cy = predT[1:2, :] + dfx_ref[8:9, :]
    a_w = predT[2:3, :] + dfx_ref[9:10, :]
    a_h = predT[3:4, :] + dfx_ref[10:11, :]

    def sl1(x):
        ax = jnp.abs(x)
        mm = jnp.minimum(ax, 1.0)
        return (ax - mm) + (0.5 * mm) * mm

    acc = jnp.zeros((8, _P_TILE), jnp.float32)
    for c in range(_G // 8):
        sl = slice(c * 8, c * 8 + 8)
        g_cx = gtx[sl, 0:1]
        g_cy = gtx[sl, 1:2]
        g_l = gtx[sl, 2:3]
        g_r = gtx[sl, 3:4]
        g_b = gtx[sl, 4:5]
        g_t = gtx[sl, 5:6]
        g_area = gtx[sl, 6:7]
        log_gw = gtx[sl, 7:8]
        log_gh = gtx[sl, 8:9]
        w = jnp.maximum(jnp.minimum(g_r, d_r) - jnp.maximum(g_l, d_l), 0.0)
        h = jnp.maximum(jnp.minimum(g_t, d_t) - jnp.maximum(g_b, d_b), 0.0)
        inter = w * h
        match = (3.0 * inter) > (g_area + d_area)
        mf_ref[sl, :] = match.astype(jnp.float32)
        x1 = a_cx - g_cx * inv_dw
        x2 = a_cy - g_cy * inv_dh
        x3 = a_w - log_gw
        x4 = a_h - log_gh
        s = sl1(x1) + sl1(x2) + sl1(x3) + sl1(x4)
        acc = acc + jnp.where(match, s, 0.0)

    lloc = jnp.sum(acc, axis=0, keepdims=True)
    mm_ = jnp.dot(gclsT, mf_ref[...], preferred_element_type=jnp.float32)
    cnt = jnp.sum(mm_, axis=0, keepdims=True)
    lcp = jnp.sum(logp * mm_, axis=0, keepdims=True)
    lc = jnp.where(cnt > 0.0, -lcp, logp[0:1, :])

    lconf_ref[0] = lc
    lloc_ref[0] = lloc
    many_ref[0] = (cnt > 0.0).astype(jnp.float32)


def _stage2_body(lconf_ref, lloc_ref, many_ref, a_ref, out_ref):
    lc = lconf_ref[...]
    f = lax.bitcast_convert_type(lc, jnp.int32)
    imin = jnp.int32(_INT_MIN)
    ukey = jnp.where(f < 0, ~f, f ^ imin)
    skey = ukey ^ imin

    posf = jnp.sum(many_ref[...], axis=1, keepdims=True)
    pos_orig = posf.astype(jnp.int32)
    pos = jnp.maximum(pos_orig, 1)
    neg = jnp.maximum(jnp.minimum(_P - pos_orig, 3 * pos), 1)
    k_lo0 = neg
    k_hi0 = _P - pos + 1

    def body(i, carry):
        prefix_lo, k_lo, prefix_hi, k_hi, hmask, bitv = carry
        masked = ukey & hmask
        bit0 = (ukey & bitv) == 0
        cand_lo = (masked == prefix_lo) & bit0
        c0_lo = jnp.sum(cand_lo.astype(jnp.int32), axis=1, keepdims=True)
        take0_lo = k_lo <= c0_lo
        prefix_lo = jnp.where(take0_lo, prefix_lo, prefix_lo | bitv)
        k_lo = jnp.where(take0_lo, k_lo, k_lo - c0_lo)
        cand_hi = (masked == prefix_hi) & bit0
        c0_hi = jnp.sum(cand_hi.astype(jnp.int32), axis=1, keepdims=True)
        take0_hi = k_hi <= c0_hi
        prefix_hi = jnp.where(take0_hi, prefix_hi, prefix_hi | bitv)
        k_hi = jnp.where(take0_hi, k_hi, k_hi - c0_hi)
        hmask = hmask | bitv
        bitv = lax.shift_right_logical(bitv, 1)
        return prefix_lo, k_lo, prefix_hi, k_hi, hmask, bitv

    zeros = jnp.zeros((_N, 1), jnp.int32)
    prefix_lo, _, prefix_hi, _, _, _ = lax.fori_loop(
        0, 32, body, (zeros, k_lo0, zeros, k_hi0, jnp.int32(0), imin),
        unroll=4)

    skth_lo = prefix_lo ^ imin
    skth_hi = prefix_hi ^ imin
    valid = (skey < skth_lo) | (skey > skth_hi)
    contrib = jnp.where(valid, lloc_ref[...] + a_ref[0, 0] * jnp.abs(lc), 0.0)
    rows = jnp.sum(contrib, axis=1, keepdims=True) / pos.astype(jnp.float32)
    out_ref[...] = jnp.sum(rows, axis=0, keepdims=True) / float(_N)


def kernel(pred_bboxes, default_bboxes, gt_bboxes, a=1):
    dcx = default_bboxes[:, 0]
    dcy = default_bboxes[:, 1]
    dw = default_bboxes[:, 2]
    dh = default_bboxes[:, 3]
    dfx = jnp.stack([
        dcx - dw * 0.5, dcx + dw * 0.5, dcy - dh * 0.5, dcy + dh * 0.5,
        dw * dh, 1.0 / dw, 1.0 / dh, dcx / dw, dcy / dh,
        jnp.log(dw), jnp.log(dh)], axis=0)

    gcx = gt_bboxes[..., 0]
    gcy = gt_bboxes[..., 1]
    gw = gt_bboxes[..., 2]
    gh = gt_bboxes[..., 3]
    gtx = jnp.concatenate([
        jnp.stack([gcx, gcy, gcx - gw * 0.5, gcx + gw * 0.5,
                   gcy - gh * 0.5, gcy + gh * 0.5, gw * gh,
                   jnp.log(gw), jnp.log(gh)], axis=-1),
        gt_bboxes[..., 4:25],
        jnp.zeros((_N, _G, 2), jnp.float32)], axis=-1)

    n_tiles = (_P + _P_TILE - 1) // _P_TILE
    out2 = jax.ShapeDtypeStruct((_N, 1, _P), jnp.float32)
    lconf, lloc, many = pl.pallas_call(
        _stage1_body,
        grid=(_N, n_tiles),
        in_specs=[
            pl.BlockSpec((1, _P_TILE, 25), lambda n, t: (n, t, 0)),
            pl.BlockSpec((1, _G, 32), lambda n, t: (n, 0, 0)),
            pl.BlockSpec((11, _P_TILE), lambda n, t: (0, t)),
        ],
        out_specs=[
            pl.BlockSpec((1, 1, _P_TILE), lambda n, t: (n, 0, t)),
            pl.BlockSpec((1, 1, _P_TILE), lambda n, t: (n, 0, t)),
            pl.BlockSpec((1, 1, _P_TILE), lambda n, t: (n, 0, t)),
        ],
        out_shape=[out2, out2, out2],
        scratch_shapes=[pltpu.VMEM((_G, _P_TILE), jnp.float32)],
    )(pred_bboxes, gtx, dfx)

    a_arr = jnp.full((1, 1), a, jnp.float32)
    out = pl.pallas_call(
        _stage2_body,
        out_shape=jax.ShapeDtypeStruct((1, 1), jnp.float32),
    )(lconf.reshape(_N, _P), lloc.reshape(_N, _P),
      many.reshape(_N, _P), a_arr)
    return out[0, 0]
